# baseline (speedup 1.0000x reference)
import jax
import jax.numpy as jnp
from jax.experimental import pallas as pl
from jax.experimental.pallas import tpu as pltpu

SEQ = 512
DIM = 128
CH = 128
NCH = SEQ // CH


def _body(pos_ref, table_hbm, out_hbm, buf, insem, outsem):
    def in_copy(j, slot):
        return pltpu.make_async_copy(
            table_hbm.at[pl.ds(pos_ref[j * CH], CH), :],
            buf.at[slot],
            insem.at[slot],
        )

    def out_copy(j, slot):
        return pltpu.make_async_copy(
            buf.at[slot],
            out_hbm.at[pl.ds(j * CH, CH), :],
            outsem.at[slot],
        )

    in_copy(0, 0).start()
    for j in range(NCH):
        slot = j % 2
        in_copy(j, slot).wait()
        out_copy(j, slot).start()
        if j + 1 < NCH:
            nslot = (j + 1) % 2
            if j >= 1:
                out_copy(j - 1, nslot).wait()
            in_copy(j + 1, nslot).start()
    out_copy(NCH - 2, NCH % 2).wait()
    out_copy(NCH - 1, (NCH - 1) % 2).wait()


def kernel(posit_embedding_weight, posit_embed_init):
    pos = posit_embed_init.astype(jnp.int32)
    out = pl.pallas_call(
        _body,
        in_specs=[
            pl.BlockSpec(memory_space=pltpu.SMEM),
            pl.BlockSpec(memory_space=pl.ANY),
        ],
        out_specs=pl.BlockSpec(memory_space=pl.ANY),
        out_shape=jax.ShapeDtypeStruct((SEQ, DIM), jnp.float32),
        scratch_shapes=[
            pltpu.VMEM((2, CH, DIM), jnp.float32),
            pltpu.SemaphoreType.DMA((2,)),
            pltpu.SemaphoreType.DMA((2,)),
        ],
    )(pos, posit_embedding_weight)
    return out[None, :, :]
